# Initial kernel scaffold; baseline (speedup 1.0000x reference)
import jax
import jax.numpy as jnp
from jax.experimental import pallas as pl


def kernel(input_ids, emb_table, fc_w, fc_b):
    raise NotImplementedError("write your pallas kernel here")


# speedup vs baseline: 1.7508x; 1.7508x over previous
import functools

import jax
import jax.numpy as jnp
from jax import lax
from jax.experimental import pallas as pl
from jax.experimental.pallas import tpu as pltpu
from jax.experimental.pallas import tpu_sc as plsc

VOCAB = 1000000
EMBED = 64
NUM_CLASSES = 100
B = 16384
L = 200

NC = 2
NS = 16
NW = NC * NS
SEQ_PER_W = B // NW
SPLIT = 128
OUT_CHUNK = 16

_mesh = plsc.VectorSubcoreMesh(core_axis_name="c", subcore_axis_name="s")


@functools.partial(
    pl.kernel,
    mesh=_mesh,
    compiler_params=pltpu.CompilerParams(use_tc_tiling_on_sc=False),
    out_type=jax.ShapeDtypeStruct((B, EMBED), jnp.float32),
    scratch_types=[
        pltpu.VMEM((L,), jnp.int32),
        pltpu.VMEM((L, EMBED), jnp.float32),
        pltpu.VMEM((OUT_CHUNK, EMBED), jnp.float32),
        pltpu.SemaphoreType.DMA,
        pltpu.SemaphoreType.DMA,
    ],
)
def _sc_pool(ids_hbm, table_hbm, out_hbm, ids_v, rows_v, pooled_v, gsem, osem):
    wid = lax.axis_index("s") * NC + lax.axis_index("c")
    seq0 = wid * SEQ_PER_W

    def group_body(g, carry):
        def seq_body(jj, carry2):
            seq = seq0 + g * OUT_CHUNK + jj
            pltpu.sync_copy(ids_hbm.at[pl.ds(L * seq, L)], ids_v)
            cp0 = pltpu.async_copy(
                table_hbm.at[ids_v.at[pl.ds(0, SPLIT)]], rows_v.at[pl.ds(0, SPLIT)], gsem)
            cp1 = pltpu.async_copy(
                table_hbm.at[ids_v.at[pl.ds(SPLIT, L - SPLIT)]],
                rows_v.at[pl.ds(SPLIT, L - SPLIT)], gsem)
            cp0.wait()
            cp1.wait()

            def red(l, accs):
                return tuple(a + rows_v[l, pl.ds(16 * i, 16)] for i, a in enumerate(accs))

            z = jnp.zeros((16,), jnp.float32)
            accs = lax.fori_loop(0, L, red, (z, z, z, z))

            scale = jnp.float32(1.0 / L)
            for i in range(4):
                pooled_v[jj, pl.ds(16 * i, 16)] = accs[i] * scale
            return carry2

        lax.fori_loop(0, OUT_CHUNK, seq_body, 0)
        base = pl.multiple_of(seq0 + g * OUT_CHUNK, OUT_CHUNK)
        pltpu.async_copy(pooled_v, out_hbm.at[pl.ds(base, OUT_CHUNK)], osem).wait()
        return carry

    lax.fori_loop(0, SEQ_PER_W // OUT_CHUNK, group_body, 0)


BM = 1024


def _fc_body(x_ref, w_ref, b_ref, o_ref):
    o_ref[...] = lax.dot_general(
        x_ref[...], w_ref[...],
        (((1,), (1,)), ((), ())),
        preferred_element_type=jnp.float32,
    ) + b_ref[...]


def _tc_fc(x, w, b2d):
    return pl.pallas_call(
        _fc_body,
        grid=(B // BM,),
        in_specs=[
            pl.BlockSpec((BM, EMBED), lambda i: (i, 0)),
            pl.BlockSpec((NUM_CLASSES, EMBED), lambda i: (0, 0)),
            pl.BlockSpec((1, NUM_CLASSES), lambda i: (0, 0)),
        ],
        out_specs=pl.BlockSpec((BM, NUM_CLASSES), lambda i: (i, 0)),
        out_shape=jax.ShapeDtypeStruct((B, NUM_CLASSES), jnp.float32),
    )(x, w, b2d)


def kernel(input_ids, emb_table, fc_w, fc_b):
    ids = input_ids.astype(jnp.int32).reshape(B * L,)
    pooled = _sc_pool(ids, emb_table)
    return _tc_fc(pooled, fc_w, fc_b.reshape(1, NUM_CLASSES))


# speedup vs baseline: 2.8032x; 1.6011x over previous
import functools

import jax
import jax.numpy as jnp
from jax import lax
from jax.experimental import pallas as pl
from jax.experimental.pallas import tpu as pltpu
from jax.experimental.pallas import tpu_sc as plsc

VOCAB = 1000000
EMBED = 64
NUM_CLASSES = 100
B = 16384
L = 200

NC = 2
NS = 16
NW = NC * NS
SEQ_PER_W = B // NW
SPLIT = 128
OUT_CHUNK = 16

_mesh = plsc.VectorSubcoreMesh(core_axis_name="c", subcore_axis_name="s")


@functools.partial(
    pl.kernel,
    mesh=_mesh,
    compiler_params=pltpu.CompilerParams(use_tc_tiling_on_sc=False),
    out_type=jax.ShapeDtypeStruct((B, EMBED), jnp.float32),
    scratch_types=[
        pltpu.VMEM((OUT_CHUNK * L,), jnp.int32),
        pltpu.VMEM((2, L, EMBED), jnp.float32),
        pltpu.VMEM((OUT_CHUNK, EMBED), jnp.float32),
        pltpu.SemaphoreType.DMA,
        pltpu.SemaphoreType.DMA,
        pltpu.SemaphoreType.DMA,
    ],
)
def _sc_pool(ids_hbm, table_hbm, out_hbm, ids_v, rows_v, pooled_v, gsem0, gsem1, osem):
    wid = lax.axis_index("s") * NC + lax.axis_index("c")
    seq0 = wid * SEQ_PER_W
    gsems = (gsem0, gsem1)

    def fire(jj, buf):
        base = L * jj
        pltpu.async_copy(
            table_hbm.at[ids_v.at[pl.ds(base, SPLIT)]],
            rows_v.at[buf].at[pl.ds(0, SPLIT)], gsems[buf])
        pltpu.async_copy(
            table_hbm.at[ids_v.at[pl.ds(base + SPLIT, L - SPLIT)]],
            rows_v.at[buf].at[pl.ds(SPLIT, L - SPLIT)], gsems[buf])

    def drain(buf):
        pltpu.make_async_copy(
            table_hbm.at[pl.ds(0, L)], rows_v.at[buf], gsems[buf]).wait()

    def reduce_into(jj, buf):
        rv = rows_v.at[buf]

        def red(l, accs):
            out = accs
            for r in range(4):
                out = tuple(
                    a + rv[4 * l + r, pl.ds(16 * i, 16)] for i, a in enumerate(out))
            return out

        z = jnp.zeros((16,), jnp.float32)
        accs = lax.fori_loop(0, L // 4, red, (z, z, z, z))
        scale = jnp.float32(1.0 / L)
        for i in range(4):
            pooled_v[jj, pl.ds(16 * i, 16)] = accs[i] * scale

    def group_body(g, carry):
        gbase = pl.multiple_of((seq0 + g * OUT_CHUNK) * L, 8)
        pltpu.sync_copy(ids_hbm.at[pl.ds(gbase, OUT_CHUNK * L)], ids_v)
        fire(0, 0)

        def pair_body(jj, carry2):
            for b in range(2):
                nxt = jj + b + 1

                @pl.when(nxt < OUT_CHUNK)
                def _fire_next():
                    fire(nxt, 1 - b)

                drain(b)
                reduce_into(jj + b, b)
            return carry2

        lax.fori_loop(0, OUT_CHUNK // 2, lambda p, c: pair_body(2 * p, c), 0)
        obase = pl.multiple_of(seq0 + g * OUT_CHUNK, 8)
        pltpu.async_copy(pooled_v, out_hbm.at[pl.ds(obase, OUT_CHUNK)], osem).wait()
        return carry

    lax.fori_loop(0, SEQ_PER_W // OUT_CHUNK, group_body, 0)


BM = 1024


def _fc_body(x_ref, w_ref, b_ref, o_ref):
    o_ref[...] = lax.dot_general(
        x_ref[...], w_ref[...],
        (((1,), (1,)), ((), ())),
        preferred_element_type=jnp.float32,
    ) + b_ref[...]


def _tc_fc(x, w, b2d):
    return pl.pallas_call(
        _fc_body,
        grid=(B // BM,),
        in_specs=[
            pl.BlockSpec((BM, EMBED), lambda i: (i, 0)),
            pl.BlockSpec((NUM_CLASSES, EMBED), lambda i: (0, 0)),
            pl.BlockSpec((1, NUM_CLASSES), lambda i: (0, 0)),
        ],
        out_specs=pl.BlockSpec((BM, NUM_CLASSES), lambda i: (i, 0)),
        out_shape=jax.ShapeDtypeStruct((B, NUM_CLASSES), jnp.float32),
    )(x, w, b2d)


def kernel(input_ids, emb_table, fc_w, fc_b):
    ids = input_ids.astype(jnp.int32).reshape(B * L,)
    pooled = _sc_pool(ids, emb_table)
    return _tc_fc(pooled, fc_w, fc_b.reshape(1, NUM_CLASSES))


# speedup vs baseline: 3.3494x; 1.1948x over previous
import functools

import jax
import jax.numpy as jnp
from jax import lax
from jax.experimental import pallas as pl
from jax.experimental.pallas import tpu as pltpu
from jax.experimental.pallas import tpu_sc as plsc

VOCAB = 1000000
EMBED = 64
NUM_CLASSES = 100
B = 16384
L = 200

NC = 2
NS = 16
NW = NC * NS
SEQ_PER_W = B // NW
SPLIT = 128
OUT_CHUNK = 64
NBUF = 4

_mesh = plsc.VectorSubcoreMesh(core_axis_name="c", subcore_axis_name="s")


@functools.partial(
    pl.kernel,
    mesh=_mesh,
    compiler_params=pltpu.CompilerParams(use_tc_tiling_on_sc=False),
    out_type=jax.ShapeDtypeStruct((B, EMBED), jnp.float32),
    scratch_types=[
        pltpu.VMEM((OUT_CHUNK * L,), jnp.int32),
        pltpu.VMEM((NBUF, L, EMBED), jnp.float32),
        pltpu.VMEM((OUT_CHUNK, EMBED), jnp.float32),
        pltpu.SemaphoreType.DMA,
        pltpu.SemaphoreType.DMA,
        pltpu.SemaphoreType.DMA,
        pltpu.SemaphoreType.DMA,
        pltpu.SemaphoreType.DMA,
    ],
)
def _sc_pool(ids_hbm, table_hbm, out_hbm, ids_v, rows_v, pooled_v,
             gsem0, gsem1, gsem2, gsem3, osem):
    wid = lax.axis_index("s") * NC + lax.axis_index("c")
    seq0 = wid * SEQ_PER_W
    gsems = (gsem0, gsem1, gsem2, gsem3)

    def fire(jj, buf):
        base = L * jj
        pltpu.async_copy(
            table_hbm.at[ids_v.at[pl.ds(base, SPLIT)]],
            rows_v.at[buf].at[pl.ds(0, SPLIT)], gsems[buf])
        pltpu.async_copy(
            table_hbm.at[ids_v.at[pl.ds(base + SPLIT, L - SPLIT)]],
            rows_v.at[buf].at[pl.ds(SPLIT, L - SPLIT)], gsems[buf])

    def drain(buf):
        pltpu.make_async_copy(
            table_hbm.at[pl.ds(0, L)], rows_v.at[buf], gsems[buf]).wait()

    def reduce_into(jj, buf):
        rv = rows_v.at[buf]

        def red(l, accs):
            out = accs
            for r in range(4):
                out = tuple(
                    a + rv[4 * l + r, pl.ds(16 * i, 16)] for i, a in enumerate(out))
            return out

        z = jnp.zeros((16,), jnp.float32)
        accs = lax.fori_loop(0, L // 4, red, (z, z, z, z))
        scale = jnp.float32(1.0 / L)
        for i in range(4):
            pooled_v[jj, pl.ds(16 * i, 16)] = accs[i] * scale

    def group_body(g, carry):
        gbase = pl.multiple_of((seq0 + g * OUT_CHUNK) * L, 8)
        pltpu.sync_copy(ids_hbm.at[pl.ds(gbase, OUT_CHUNK * L)], ids_v)
        for b in range(NBUF - 1):
            fire(b, b)

        def ring_body(jj, carry2):
            for b in range(NBUF):
                nxt = jj + b + (NBUF - 1)

                @pl.when(nxt < OUT_CHUNK)
                def _fire_next():
                    fire(nxt, (b + NBUF - 1) % NBUF)

                drain(b)
                reduce_into(jj + b, b)
            return carry2

        lax.fori_loop(0, OUT_CHUNK // NBUF, lambda p, c: ring_body(NBUF * p, c), 0)
        obase = pl.multiple_of(seq0 + g * OUT_CHUNK, 8)
        pltpu.async_copy(pooled_v, out_hbm.at[pl.ds(obase, OUT_CHUNK)], osem).wait()
        return carry

    lax.fori_loop(0, SEQ_PER_W // OUT_CHUNK, group_body, 0)


BM = 1024


def _fc_body(x_ref, w_ref, b_ref, o_ref):
    o_ref[...] = lax.dot_general(
        x_ref[...], w_ref[...],
        (((1,), (1,)), ((), ())),
        preferred_element_type=jnp.float32,
    ) + b_ref[...]


def _tc_fc(x, w, b2d):
    return pl.pallas_call(
        _fc_body,
        grid=(B // BM,),
        in_specs=[
            pl.BlockSpec((BM, EMBED), lambda i: (i, 0)),
            pl.BlockSpec((NUM_CLASSES, EMBED), lambda i: (0, 0)),
            pl.BlockSpec((1, NUM_CLASSES), lambda i: (0, 0)),
        ],
        out_specs=pl.BlockSpec((BM, NUM_CLASSES), lambda i: (i, 0)),
        out_shape=jax.ShapeDtypeStruct((B, NUM_CLASSES), jnp.float32),
    )(x, w, b2d)


def kernel(input_ids, emb_table, fc_w, fc_b):
    ids = input_ids.astype(jnp.int32).reshape(B * L,)
    pooled = _sc_pool(ids, emb_table)
    return _tc_fc(pooled, fc_w, fc_b.reshape(1, NUM_CLASSES))


# speedup vs baseline: 3.3549x; 1.0017x over previous
import functools

import jax
import jax.numpy as jnp
from jax import lax
from jax.experimental import pallas as pl
from jax.experimental.pallas import tpu as pltpu
from jax.experimental.pallas import tpu_sc as plsc

VOCAB = 1000000
EMBED = 64
NUM_CLASSES = 100
B = 16384
L = 200

NC = 2
NS = 16
NW = NC * NS
SEQ_PER_W = B // NW
SPLIT = 128
OUT_CHUNK = 64
NBUF = 4

_mesh = plsc.VectorSubcoreMesh(core_axis_name="c", subcore_axis_name="s")


@functools.partial(
    pl.kernel,
    mesh=_mesh,
    compiler_params=pltpu.CompilerParams(use_tc_tiling_on_sc=False),
    out_type=jax.ShapeDtypeStruct((B, EMBED), jnp.float32),
    scratch_types=[
        pltpu.VMEM((OUT_CHUNK * L,), jnp.int32),
        pltpu.VMEM((NBUF, L, EMBED), jnp.float32),
        pltpu.VMEM((OUT_CHUNK, EMBED), jnp.float32),
        pltpu.SemaphoreType.DMA,
        pltpu.SemaphoreType.DMA,
        pltpu.SemaphoreType.DMA,
        pltpu.SemaphoreType.DMA,
        pltpu.SemaphoreType.DMA,
    ],
)
def _sc_pool(ids_hbm, table_hbm, out_hbm, ids_v, rows_v, pooled_v,
             gsem0, gsem1, gsem2, gsem3, osem):
    wid = lax.axis_index("s") * NC + lax.axis_index("c")
    seq0 = wid * SEQ_PER_W
    gsems = (gsem0, gsem1, gsem2, gsem3)

    def fire(jj, buf):
        base = L * jj
        pltpu.async_copy(
            table_hbm.at[ids_v.at[pl.ds(base, SPLIT)]],
            rows_v.at[buf].at[pl.ds(0, SPLIT)], gsems[buf])
        pltpu.async_copy(
            table_hbm.at[ids_v.at[pl.ds(base + SPLIT, L - SPLIT)]],
            rows_v.at[buf].at[pl.ds(SPLIT, L - SPLIT)], gsems[buf])

    def drain(buf):
        pltpu.make_async_copy(
            table_hbm.at[pl.ds(0, L)], rows_v.at[buf], gsems[buf]).wait()

    ROWS_PER_ITER = 25

    def reduce_into(jj, buf):
        rv = rows_v.at[buf]

        def red(l, accs):
            out = list(accs)
            for r in range(ROWS_PER_ITER):
                row = ROWS_PER_ITER * l + r
                for i in range(4):
                    k = 4 * (r % 2) + i
                    out[k] = out[k] + rv[row, pl.ds(16 * i, 16)]
            return tuple(out)

        z = jnp.zeros((16,), jnp.float32)
        accs = lax.fori_loop(0, L // ROWS_PER_ITER, red, (z,) * 8)
        scale = jnp.float32(1.0 / L)
        for i in range(4):
            pooled_v[jj, pl.ds(16 * i, 16)] = (accs[i] + accs[4 + i]) * scale

    def group_body(g, carry):
        gbase = pl.multiple_of((seq0 + g * OUT_CHUNK) * L, 8)
        pltpu.sync_copy(ids_hbm.at[pl.ds(gbase, OUT_CHUNK * L)], ids_v)
        for b in range(NBUF - 1):
            fire(b, b)

        def ring_body(jj, carry2):
            for b in range(NBUF):
                nxt = jj + b + (NBUF - 1)

                @pl.when(nxt < OUT_CHUNK)
                def _fire_next():
                    fire(nxt, (b + NBUF - 1) % NBUF)

                drain(b)
                reduce_into(jj + b, b)
            return carry2

        lax.fori_loop(0, OUT_CHUNK // NBUF, lambda p, c: ring_body(NBUF * p, c), 0)
        obase = pl.multiple_of(seq0 + g * OUT_CHUNK, 8)
        pltpu.async_copy(pooled_v, out_hbm.at[pl.ds(obase, OUT_CHUNK)], osem).wait()
        return carry

    lax.fori_loop(0, SEQ_PER_W // OUT_CHUNK, group_body, 0)


BM = 1024


def _fc_body(x_ref, w_ref, b_ref, o_ref):
    o_ref[...] = lax.dot_general(
        x_ref[...], w_ref[...],
        (((1,), (1,)), ((), ())),
        preferred_element_type=jnp.float32,
    ) + b_ref[...]


def _tc_fc(x, w, b2d):
    return pl.pallas_call(
        _fc_body,
        grid=(B // BM,),
        in_specs=[
            pl.BlockSpec((BM, EMBED), lambda i: (i, 0)),
            pl.BlockSpec((NUM_CLASSES, EMBED), lambda i: (0, 0)),
            pl.BlockSpec((1, NUM_CLASSES), lambda i: (0, 0)),
        ],
        out_specs=pl.BlockSpec((BM, NUM_CLASSES), lambda i: (i, 0)),
        out_shape=jax.ShapeDtypeStruct((B, NUM_CLASSES), jnp.float32),
    )(x, w, b2d)


def kernel(input_ids, emb_table, fc_w, fc_b):
    ids = input_ids.astype(jnp.int32).reshape(B * L,)
    pooled = _sc_pool(ids, emb_table)
    return _tc_fc(pooled, fc_w, fc_b.reshape(1, NUM_CLASSES))
